# baseline (speedup 1.0000x reference)
import functools

import jax
import jax.numpy as jnp
from jax import lax
from jax.experimental import pallas as pl
from jax.experimental.pallas import tpu as pltpu
from jax.experimental.pallas import tpu_sc as plsc

N = 16384
C = 128
V = 64
NP = N // 2
PAIRS = V * V
SUBP = 64


def _ptable_body(inp1_ref, tbl_ref):
    x = inp1_ref[...]
    xt = jnp.transpose(x, (1, 0))
    sq = xt[:V] * xt[:V]
    left = jnp.broadcast_to(sq[:, None, :], (V, V, C)).reshape(PAIRS, C)
    right = jnp.broadcast_to(sq[None, :, :], (V, V, C)).reshape(PAIRS, C)
    tbl_ref[:, :C] = left
    tbl_ref[:, C:] = right


def _build_ptable(inp1):
    return pl.pallas_call(
        _ptable_body,
        grid=(1,),
        in_specs=[pl.BlockSpec((C, 128), lambda j: (0, 0))],
        out_specs=pl.BlockSpec((PAIRS, 2 * C), lambda j: (0, 0)),
        out_shape=jax.ShapeDtypeStruct((PAIRS, 2 * C), jnp.float32),
    )(inp1)


@functools.cache
def _make_sc_gather():
    info = plsc.get_sparse_core_info()
    nc, ns, nl = info.num_cores, info.num_subcores, info.num_lanes
    nw = nc * ns
    i_per_w = N // nw
    p_per_w = NP // nw
    n_sub = p_per_w // SUBP
    mesh = plsc.VectorSubcoreMesh(core_axis_name="c", subcore_axis_name="s")

    @functools.partial(
        pl.kernel,
        mesh=mesh,
        out_type=jax.ShapeDtypeStruct((NP, 2 * C), jnp.float32),
        compiler_params=pltpu.CompilerParams(needs_layout_passes=False),
        scratch_types=[
            pltpu.VMEM((i_per_w,), jnp.int32),
            pltpu.VMEM((p_per_w,), jnp.int32),
            [pltpu.VMEM((SUBP, 2 * C), jnp.float32) for _ in range(n_sub)],
            [pltpu.SemaphoreType.DMA for _ in range(n_sub)],
            [pltpu.SemaphoreType.DMA for _ in range(n_sub)],
        ],
    )
    def gather_k(tbl_hbm, idx_hbm, out_hbm, idx_v, pidx_v, rows, gsems, ssems):
        wid = lax.axis_index("s") * nc + lax.axis_index("c")
        base = wid * i_per_w
        pltpu.sync_copy(idx_hbm.at[pl.ds(base, i_per_w)], idx_v)
        iota2 = jnp.arange(nl, dtype=jnp.int32) * 2
        for i in range(p_per_w // nl):
            g0 = i * nl * 2
            a = plsc.load_gather(idx_v, [g0 + iota2])
            b = plsc.load_gather(idx_v, [g0 + iota2 + 1])
            a = jnp.minimum(jnp.maximum(a, 0), V - 1)
            b = jnp.minimum(jnp.maximum(b, 0), V - 1)
            pidx_v[pl.ds(i * nl, nl)] = a * V + b
        gs = [
            pltpu.async_copy(
                tbl_hbm.at[pidx_v.at[pl.ds(k * SUBP, SUBP)]], rows[k], gsems[k]
            )
            for k in range(n_sub)
        ]
        ss = []
        for k in range(n_sub):
            gs[k].wait()
            ss.append(
                pltpu.async_copy(
                    rows[k],
                    out_hbm.at[pl.ds(wid * p_per_w + k * SUBP, SUBP)],
                    ssems[k],
                )
            )
        for s in ss:
            s.wait()

    return gather_k


_BLK = 16384


def _mul_body(inp1_ref, g_ref, o_ref):
    g2 = g_ref[...].reshape(_BLK, C)
    o_ref[...] = jnp.transpose(inp1_ref[...], (1, 0)) * g2


def _mul(inp1, g):
    return pl.pallas_call(
        _mul_body,
        grid=(N // _BLK,),
        in_specs=[
            pl.BlockSpec((C, _BLK), lambda j: (0, j)),
            pl.BlockSpec((_BLK // 2, 2 * C), lambda j: (j, 0)),
        ],
        out_specs=pl.BlockSpec((_BLK, C), lambda j: (j, 0)),
        out_shape=jax.ShapeDtypeStruct((N, C), jnp.float32),
    )(inp1, g)


def kernel(inp1, inp2):
    idx = inp2.reshape(N).astype(jnp.int32)
    ptable = _build_ptable(inp1)
    g = _make_sc_gather()(ptable, idx)
    out = _mul(inp1, g)
    return (out,)


# speedup vs baseline: 1.0667x; 1.0667x over previous
import functools

import jax
import jax.numpy as jnp
from jax import lax
from jax.experimental import pallas as pl
from jax.experimental.pallas import tpu as pltpu
from jax.experimental.pallas import tpu_sc as plsc

N = 16384
C = 128
V = 64
NP = N // 2
PAIRS = V * V
SUBP = 64


def _ptable_body(inp1_ref, tbl_ref):
    x = inp1_ref[...]
    xt = jnp.transpose(x, (1, 0))
    sq = xt[:V] * xt[:V]
    left = jnp.broadcast_to(sq[:, None, :], (V, V, C)).reshape(PAIRS, C)
    right = jnp.broadcast_to(sq[None, :, :], (V, V, C)).reshape(PAIRS, C)
    tbl_ref[:, :C] = left
    tbl_ref[:, C:] = right


def _build_ptable(inp1):
    return pl.pallas_call(
        _ptable_body,
        grid=(1,),
        in_specs=[pl.BlockSpec((C, 128), lambda j: (0, 0))],
        out_specs=pl.BlockSpec((PAIRS, 2 * C), lambda j: (0, 0)),
        out_shape=jax.ShapeDtypeStruct((PAIRS, 2 * C), jnp.float32),
    )(inp1)


@functools.cache
def _make_sc_gather():
    info = plsc.get_sparse_core_info()
    nc, ns, nl = info.num_cores, info.num_subcores, info.num_lanes
    nw = nc * ns
    i_per_w = N // nw
    p_per_w = NP // nw
    n_sub = p_per_w // SUBP
    mesh = plsc.VectorSubcoreMesh(core_axis_name="c", subcore_axis_name="s")

    @functools.partial(
        pl.kernel,
        mesh=mesh,
        out_type=jax.ShapeDtypeStruct((NP, 2 * C), jnp.float32),
        compiler_params=pltpu.CompilerParams(needs_layout_passes=False),
        scratch_types=[
            pltpu.VMEM((i_per_w,), jnp.int32),
            pltpu.VMEM((p_per_w,), jnp.int32),
            [pltpu.VMEM((SUBP, 2 * C), jnp.float32) for _ in range(n_sub)],
            [pltpu.SemaphoreType.DMA for _ in range(n_sub)],
            [pltpu.SemaphoreType.DMA for _ in range(n_sub)],
        ],
    )
    def gather_k(tbl_hbm, idx_hbm, out_hbm, idx_v, pidx_v, rows, gsems, ssems):
        wid = lax.axis_index("s") * nc + lax.axis_index("c")
        base = wid * i_per_w
        pltpu.sync_copy(idx_hbm.at[pl.ds(base, i_per_w)], idx_v)
        iota2 = jnp.arange(nl, dtype=jnp.int32) * 2
        for i in range(p_per_w // nl):
            g0 = i * nl * 2
            a = plsc.load_gather(idx_v, [g0 + iota2])
            b = plsc.load_gather(idx_v, [g0 + iota2 + 1])
            a = jnp.minimum(jnp.maximum(a, 0), V - 1)
            b = jnp.minimum(jnp.maximum(b, 0), V - 1)
            pidx_v[pl.ds(i * nl, nl)] = a * V + b
        gs = [
            pltpu.async_copy(
                tbl_hbm.at[pidx_v.at[pl.ds(k * SUBP, SUBP)]], rows[k], gsems[k]
            )
            for k in range(n_sub)
        ]
        ss = []
        for k in range(n_sub):
            gs[k].wait()
            ss.append(
                pltpu.async_copy(
                    rows[k],
                    out_hbm.at[pl.ds(wid * p_per_w + k * SUBP, SUBP)],
                    ssems[k],
                )
            )
        for s in ss:
            s.wait()

    return gather_k


_BLK = 8192


def _mul_body(inp1_ref, g_ref, o_ref):
    g2 = g_ref[...].reshape(_BLK, C)
    o_ref[...] = jnp.transpose(inp1_ref[...], (1, 0)) * g2


def _mul(inp1, g):
    return pl.pallas_call(
        _mul_body,
        grid=(N // _BLK,),
        in_specs=[
            pl.BlockSpec((C, _BLK), lambda j: (0, j)),
            pl.BlockSpec((_BLK // 2, 2 * C), lambda j: (j, 0)),
        ],
        out_specs=pl.BlockSpec((_BLK, C), lambda j: (j, 0)),
        out_shape=jax.ShapeDtypeStruct((N, C), jnp.float32),
    )(inp1, g)


def kernel(inp1, inp2):
    idx = inp2.reshape(N).astype(jnp.int32)
    ptable = _build_ptable(inp1)
    g = _make_sc_gather()(ptable, idx)
    out = _mul(inp1, g)
    return (out,)
